# Initial kernel scaffold; baseline (speedup 1.0000x reference)
import jax
import jax.numpy as jnp
from jax.experimental import pallas as pl


def kernel(x, edge_index, w1, b1, w2, b2, wfc, bfc):
    raise NotImplementedError("write your pallas kernel here")


# speedup vs baseline: 85.0804x; 85.0804x over previous
import functools

import jax
import jax.numpy as jnp
from jax import lax
from jax.experimental import pallas as pl
from jax.experimental.pallas import tpu as pltpu
from jax.experimental.pallas import tpu_sc as plsc

N = 10000
NP = 10240
E = 640000
F = 16
NC = 2
NS = 16
NW = NC * NS
E_PER_TILE = E // NW
CH = 2000
N_CHUNKS = E_PER_TILE // CH
N_PER_TILE = NP // NS


def _mesh():
    return plsc.VectorSubcoreMesh(core_axis_name="c", subcore_axis_name="s")


@functools.partial(
    pl.kernel,
    out_type=jax.ShapeDtypeStruct((NC, NP, F), jnp.float32),
    mesh=_mesh(),
    compiler_params=pltpu.CompilerParams(use_tc_tiling_on_sc=False),
    scratch_types=[
        pltpu.VMEM_SHARED((NP, F), jnp.float32),
        pltpu.VMEM((CH,), jnp.int32),
        pltpu.VMEM((CH, F), jnp.float32),
    ],
)
def _sc_degree(col_hbm, ones_hbm, zeros_hbm, out_hbm, acc_sp, idx_v, ones_v):
    cid = lax.axis_index("c")
    sid = lax.axis_index("s")
    wid = cid * NS + sid
    rbase = sid * N_PER_TILE
    pltpu.sync_copy(zeros_hbm.at[pl.ds(rbase, N_PER_TILE)],
                    acc_sp.at[pl.ds(rbase, N_PER_TILE)])
    pltpu.sync_copy(ones_hbm, ones_v)
    plsc.subcore_barrier()

    def step(k, carry):
        base = wid * E_PER_TILE + k * CH
        pltpu.sync_copy(col_hbm.at[pl.ds(base, CH)], idx_v)
        pltpu.sync_copy(ones_v, acc_sp.at[idx_v], add=True)
        return carry

    lax.fori_loop(0, N_CHUNKS, step, 0)
    plsc.subcore_barrier()
    pltpu.sync_copy(acc_sp.at[pl.ds(rbase, N_PER_TILE)],
                    out_hbm.at[cid, pl.ds(rbase, N_PER_TILE)])


@functools.partial(
    pl.kernel,
    out_type=jax.ShapeDtypeStruct((NC, NP, F), jnp.float32),
    mesh=_mesh(),
    compiler_params=pltpu.CompilerParams(use_tc_tiling_on_sc=False),
    scratch_types=[
        pltpu.VMEM_SHARED((NP, F), jnp.float32),
        pltpu.VMEM_SHARED((NP, F), jnp.float32),
        pltpu.VMEM((CH,), jnp.int32),
        pltpu.VMEM((CH,), jnp.int32),
        pltpu.VMEM((CH, F), jnp.float32),
    ],
)
def _sc_msgpass(g_hbm, row_hbm, col_hbm, zeros_hbm, out_hbm,
                g_sp, acc_sp, rows_v, cols_v, msgs_v):
    cid = lax.axis_index("c")
    sid = lax.axis_index("s")
    wid = cid * NS + sid
    rbase = sid * N_PER_TILE
    pltpu.sync_copy(zeros_hbm.at[pl.ds(rbase, N_PER_TILE)],
                    acc_sp.at[pl.ds(rbase, N_PER_TILE)])
    pltpu.sync_copy(g_hbm.at[pl.ds(rbase, N_PER_TILE)],
                    g_sp.at[pl.ds(rbase, N_PER_TILE)])
    plsc.subcore_barrier()

    def step(k, carry):
        base = wid * E_PER_TILE + k * CH
        pltpu.sync_copy(row_hbm.at[pl.ds(base, CH)], rows_v)
        pltpu.sync_copy(col_hbm.at[pl.ds(base, CH)], cols_v)
        pltpu.sync_copy(g_sp.at[rows_v], msgs_v)
        pltpu.sync_copy(msgs_v, acc_sp.at[cols_v], add=True)
        return carry

    lax.fori_loop(0, N_CHUNKS, step, 0)
    plsc.subcore_barrier()
    pltpu.sync_copy(acc_sp.at[pl.ds(rbase, N_PER_TILE)],
                    out_hbm.at[cid, pl.ds(rbase, N_PER_TILE)])


def _tc1_body(x_ref, w1_ref, degp_ref, g1_ref, dinv_ref):
    deg = degp_ref[0] + degp_ref[1] + 1.0
    dinv = lax.rsqrt(deg)
    h = jnp.dot(x_ref[...], w1_ref[...], preferred_element_type=jnp.float32)
    g1_ref[...] = dinv * h
    dinv_ref[...] = dinv


def _tc2_body(acc_ref, g1_ref, dinv_ref, b1_ref, w2p_ref, g2_ref):
    s = acc_ref[0] + acc_ref[1] + g1_ref[...]
    h1 = jnp.maximum(dinv_ref[...] * s + b1_ref[...], 0.0)
    h2 = jnp.dot(h1, w2p_ref[...], preferred_element_type=jnp.float32)
    g2_ref[...] = dinv_ref[...] * h2


def _tc3_body(acc_ref, g2_ref, dinv_ref, b2p_ref, wfcs_ref, bfc_ref, out_ref):
    s = acc_ref[0] + acc_ref[1] + g2_ref[...]
    h = jnp.maximum(dinv_ref[...] * s + b2p_ref[...], 0.0)
    prod = h[None, :, :] * wfcs_ref[...]
    sums = jnp.sum(prod, axis=(1, 2))
    out_ref[...] = sums.reshape(1, 2) + bfc_ref[...]


def kernel(x, edge_index, w1, b1, w2, b2, wfc, bfc):
    row = edge_index[0]
    col = edge_index[1]
    ones_ch = jnp.ones((CH, F), jnp.float32)
    zeros_n = jnp.zeros((NP, F), jnp.float32)

    degp = _sc_degree(col, ones_ch, zeros_n)

    xp = jnp.pad(x, ((0, NP - N), (0, 0)))
    g1, dinv = pl.pallas_call(
        _tc1_body,
        out_shape=(jax.ShapeDtypeStruct((NP, F), jnp.float32),
                   jax.ShapeDtypeStruct((NP, F), jnp.float32)),
    )(xp, w1, degp)

    acc1 = _sc_msgpass(g1, row, col, zeros_n)

    w2p = jnp.pad(w2, ((0, 0), (0, F - 8)))
    g2 = pl.pallas_call(
        _tc2_body,
        out_shape=jax.ShapeDtypeStruct((NP, F), jnp.float32),
    )(acc1, g1, dinv, b1.reshape(1, F), w2p)

    acc2 = _sc_msgpass(g2, row, col, zeros_n)

    wfcs = jnp.pad(wfc.reshape(N, 8, 2).transpose(2, 0, 1),
                   ((0, 0), (0, NP - N), (0, F - 8)))
    b2p = jnp.pad(b2, (0, F - 8)).reshape(1, F)
    out = pl.pallas_call(
        _tc3_body,
        out_shape=jax.ShapeDtypeStruct((1, 2), jnp.float32),
    )(acc2, g2, dinv, b2p, wfcs, bfc.reshape(1, 2))
    return out


# speedup vs baseline: 92.8413x; 1.0912x over previous
import functools

import jax
import jax.numpy as jnp
from jax import lax
from jax.experimental import pallas as pl
from jax.experimental.pallas import tpu as pltpu
from jax.experimental.pallas import tpu_sc as plsc

N = 10000
NP = 10240
E = 640000
NC = 2
NS = 16
NW = NC * NS
E_PER_TILE = E // NW
CH = 2000
N_CHUNKS = E_PER_TILE // CH
N_PER_TILE = NP // NS
FD = 8


def _mesh():
    return plsc.VectorSubcoreMesh(core_axis_name="c", subcore_axis_name="s")


_SC_PARAMS = pltpu.CompilerParams(use_tc_tiling_on_sc=False)


@functools.partial(
    pl.kernel,
    out_type=jax.ShapeDtypeStruct((NC, NP, FD), jnp.float32),
    mesh=_mesh(),
    compiler_params=_SC_PARAMS,
    scratch_types=[
        pltpu.VMEM_SHARED((NP, FD), jnp.float32),
        pltpu.VMEM((CH,), jnp.int32),
        pltpu.VMEM((CH, FD), jnp.float32),
    ],
)
def _sc_degree(col_hbm, ones_hbm, zeros_hbm, out_hbm, acc_sp, idx_v, ones_v):
    cid = lax.axis_index("c")
    sid = lax.axis_index("s")
    wid = cid * NS + sid
    rbase = sid * N_PER_TILE
    pltpu.sync_copy(zeros_hbm.at[pl.ds(rbase, N_PER_TILE)],
                    acc_sp.at[pl.ds(rbase, N_PER_TILE)])
    pltpu.sync_copy(ones_hbm, ones_v)
    plsc.subcore_barrier()

    def step(k, carry):
        base = wid * E_PER_TILE + k * CH
        pltpu.sync_copy(col_hbm.at[pl.ds(base, CH)], idx_v)
        pltpu.sync_copy(ones_v, acc_sp.at[idx_v], add=True)
        return carry

    lax.fori_loop(0, N_CHUNKS, step, 0)
    plsc.subcore_barrier()
    pltpu.sync_copy(acc_sp.at[pl.ds(rbase, N_PER_TILE)],
                    out_hbm.at[cid, pl.ds(rbase, N_PER_TILE)])


def _make_msgpass(f):
    @functools.partial(
        pl.kernel,
        out_type=jax.ShapeDtypeStruct((NC, NP, f), jnp.float32),
        mesh=_mesh(),
        compiler_params=_SC_PARAMS,
        scratch_types=[
            pltpu.VMEM_SHARED((NP, f), jnp.float32),
            pltpu.VMEM_SHARED((NP, f), jnp.float32),
            pltpu.VMEM((CH,), jnp.int32),
            pltpu.VMEM((CH,), jnp.int32),
            pltpu.VMEM((CH, f), jnp.float32),
        ],
    )
    def _msgpass(g_hbm, row_hbm, col_hbm, zeros_hbm, out_hbm,
                 g_sp, acc_sp, rows_v, cols_v, msgs_v):
        cid = lax.axis_index("c")
        sid = lax.axis_index("s")
        wid = cid * NS + sid
        rbase = sid * N_PER_TILE
        pltpu.sync_copy(zeros_hbm.at[pl.ds(rbase, N_PER_TILE)],
                        acc_sp.at[pl.ds(rbase, N_PER_TILE)])
        pltpu.sync_copy(g_hbm.at[pl.ds(rbase, N_PER_TILE)],
                        g_sp.at[pl.ds(rbase, N_PER_TILE)])
        plsc.subcore_barrier()

        def step(k, carry):
            base = wid * E_PER_TILE + k * CH
            pltpu.sync_copy(row_hbm.at[pl.ds(base, CH)], rows_v)
            pltpu.sync_copy(col_hbm.at[pl.ds(base, CH)], cols_v)
            pltpu.sync_copy(g_sp.at[rows_v], msgs_v)
            pltpu.sync_copy(msgs_v, acc_sp.at[cols_v], add=True)
            return carry

        lax.fori_loop(0, N_CHUNKS, step, 0)
        plsc.subcore_barrier()
        pltpu.sync_copy(acc_sp.at[pl.ds(rbase, N_PER_TILE)],
                        out_hbm.at[cid, pl.ds(rbase, N_PER_TILE)])

    return _msgpass


_msgpass16 = _make_msgpass(16)
_msgpass8 = _make_msgpass(8)


def _mm1_body(x_ref, w1_ref, h1_ref):
    h = jnp.dot(x_ref[...], w1_ref[...], preferred_element_type=jnp.float32)
    h1_ref[:N] = h
    h1_ref[N:] = jnp.zeros((NP - N, 16), jnp.float32)


def _scale1_body(degp_ref, h1_ref, g1_ref, dinv_ref):
    deg8 = degp_ref[0] + degp_ref[1] + 1.0
    dinv8 = lax.rsqrt(deg8)
    dinv16 = jnp.concatenate([dinv8, dinv8], axis=-1)
    g1_ref[...] = dinv16 * h1_ref[...]
    dinv_ref[...] = dinv16


def _tc2_body(acc_ref, g1_ref, dinv_ref, b1_ref, w2_ref, g2_ref):
    s = acc_ref[0] + acc_ref[1] + g1_ref[...]
    h1 = jnp.maximum(dinv_ref[...] * s + b1_ref[...], 0.0)
    h2 = jnp.dot(h1, w2_ref[...], preferred_element_type=jnp.float32)
    g2_ref[...] = dinv_ref[:, :8] * h2


def _tc3_body(acc_ref, g2_ref, dinv_ref, b2_ref, wfcs_ref, bfc_ref, out_ref):
    s = acc_ref[0] + acc_ref[1] + g2_ref[...]
    h = jnp.maximum(dinv_ref[:, :8] * s + b2_ref[...], 0.0)
    prod = h[None, :, :] * wfcs_ref[...]
    sums = jnp.sum(prod, axis=(1, 2))
    out_ref[...] = sums.reshape(1, 2) + bfc_ref[...]


def kernel(x, edge_index, w1, b1, w2, b2, wfc, bfc):
    row = edge_index[0]
    col = edge_index[1]
    ones_d = jnp.ones((CH, FD), jnp.float32)
    zeros8 = jnp.zeros((NP, FD), jnp.float32)
    zeros16 = jnp.zeros((NP, 16), jnp.float32)

    degp = _sc_degree(col, ones_d, zeros8)

    h1 = pl.pallas_call(
        _mm1_body,
        out_shape=jax.ShapeDtypeStruct((NP, 16), jnp.float32),
    )(x, w1)

    g1, dinv16 = pl.pallas_call(
        _scale1_body,
        out_shape=(jax.ShapeDtypeStruct((NP, 16), jnp.float32),
                   jax.ShapeDtypeStruct((NP, 16), jnp.float32)),
    )(degp, h1)

    acc1 = _msgpass16(g1, row, col, zeros16)

    g2 = pl.pallas_call(
        _tc2_body,
        out_shape=jax.ShapeDtypeStruct((NP, 8), jnp.float32),
    )(acc1, g1, dinv16, b1.reshape(1, 16), w2)

    acc2 = _msgpass8(g2, row, col, zeros8)

    wfcs = jnp.pad(wfc.reshape(N, 8, 2).transpose(2, 0, 1),
                   ((0, 0), (0, NP - N), (0, 0)))
    out = pl.pallas_call(
        _tc3_body,
        out_shape=jax.ShapeDtypeStruct((1, 2), jnp.float32),
    )(acc2, g2, dinv16, b2.reshape(1, 8), wfcs, bfc.reshape(1, 2))
    return out


# speedup vs baseline: 96.0232x; 1.0343x over previous
import functools

import jax
import jax.numpy as jnp
from jax import lax
from jax.experimental import pallas as pl
from jax.experimental.pallas import tpu as pltpu
from jax.experimental.pallas import tpu_sc as plsc

N = 10000
NP = 10240
E = 640000
NC = 2
NS = 16
NW = NC * NS
E_PER_TILE = E // NW
CH = 2000
N_CHUNKS = E_PER_TILE // CH
N_PER_TILE = NP // NS
FD = 8


def _mesh():
    return plsc.VectorSubcoreMesh(core_axis_name="c", subcore_axis_name="s")


_SC_PARAMS = pltpu.CompilerParams(use_tc_tiling_on_sc=False)


@functools.partial(
    pl.kernel,
    out_type=jax.ShapeDtypeStruct((NC, NP, FD), jnp.float32),
    mesh=_mesh(),
    compiler_params=_SC_PARAMS,
    scratch_types=[
        pltpu.VMEM_SHARED((NP, FD), jnp.float32),
        pltpu.VMEM((CH,), jnp.int32),
        pltpu.VMEM((CH, FD), jnp.float32),
    ],
)
def _sc_degree(ei_hbm, ones_hbm, zeros_hbm, out_hbm, acc_sp, idx_v, ones_v):
    cid = lax.axis_index("c")
    sid = lax.axis_index("s")
    wid = cid * NS + sid
    rbase = sid * N_PER_TILE
    pltpu.sync_copy(zeros_hbm.at[pl.ds(rbase, N_PER_TILE)],
                    acc_sp.at[pl.ds(rbase, N_PER_TILE)])
    pltpu.sync_copy(ones_hbm, ones_v)
    plsc.subcore_barrier()

    def step(k, carry):
        base = wid * E_PER_TILE + k * CH
        pltpu.sync_copy(ei_hbm.at[1, pl.ds(base, CH)], idx_v)
        pltpu.sync_copy(ones_v, acc_sp.at[idx_v], add=True)
        return carry

    lax.fori_loop(0, N_CHUNKS, step, 0)
    plsc.subcore_barrier()
    pltpu.sync_copy(acc_sp.at[pl.ds(rbase, N_PER_TILE)],
                    out_hbm.at[cid, pl.ds(rbase, N_PER_TILE)])


def _make_msgpass(f):
    @functools.partial(
        pl.kernel,
        out_type=jax.ShapeDtypeStruct((NC, NP, f), jnp.float32),
        mesh=_mesh(),
        compiler_params=_SC_PARAMS,
        scratch_types=[
            pltpu.VMEM_SHARED((NP, f), jnp.float32),
            pltpu.VMEM_SHARED((NP, f), jnp.float32),
            pltpu.VMEM((CH,), jnp.int32),
            pltpu.VMEM((CH,), jnp.int32),
            pltpu.VMEM((CH, f), jnp.float32),
        ],
    )
    def _msgpass(g_hbm, ei_hbm, zeros_hbm, out_hbm,
                 g_sp, acc_sp, rows_v, cols_v, msgs_v):
        cid = lax.axis_index("c")
        sid = lax.axis_index("s")
        wid = cid * NS + sid
        rbase = sid * N_PER_TILE
        pltpu.sync_copy(zeros_hbm.at[pl.ds(rbase, N_PER_TILE)],
                        acc_sp.at[pl.ds(rbase, N_PER_TILE)])
        pltpu.sync_copy(g_hbm.at[pl.ds(rbase, N_PER_TILE)],
                        g_sp.at[pl.ds(rbase, N_PER_TILE)])
        plsc.subcore_barrier()

        def step(k, carry):
            base = wid * E_PER_TILE + k * CH
            pltpu.sync_copy(ei_hbm.at[0, pl.ds(base, CH)], rows_v)
            pltpu.sync_copy(ei_hbm.at[1, pl.ds(base, CH)], cols_v)
            pltpu.sync_copy(g_sp.at[rows_v], msgs_v)
            pltpu.sync_copy(msgs_v, acc_sp.at[cols_v], add=True)
            return carry

        lax.fori_loop(0, N_CHUNKS, step, 0)
        plsc.subcore_barrier()
        pltpu.sync_copy(acc_sp.at[pl.ds(rbase, N_PER_TILE)],
                        out_hbm.at[cid, pl.ds(rbase, N_PER_TILE)])

    return _msgpass


_msgpass16 = _make_msgpass(16)
_msgpass8 = _make_msgpass(8)


def _tc1_body(x_ref, w1_ref, degp_ref, g1_ref, dinv_ref):
    deg8 = degp_ref[0] + degp_ref[1] + 1.0
    dinv8 = lax.rsqrt(deg8)
    dinv16 = jnp.concatenate([dinv8, dinv8], axis=-1)
    h = jnp.dot(x_ref[...], w1_ref[...], preferred_element_type=jnp.float32)
    g1_ref[:N] = dinv16[:N] * h
    g1_ref[N:] = jnp.zeros((NP - N, 16), jnp.float32)
    dinv_ref[...] = dinv16


def _tc2_body(acc_ref, g1_ref, dinv_ref, b1_ref, w2_ref, g2_ref):
    s = acc_ref[0] + acc_ref[1] + g1_ref[...]
    h1 = jnp.maximum(dinv_ref[...] * s + b1_ref[...], 0.0)
    h2 = jnp.dot(h1, w2_ref[...], preferred_element_type=jnp.float32)
    g2_ref[...] = dinv_ref[:, :8] * h2


def _tc3_body(acc_ref, g2_ref, dinv_ref, b2_ref, wfcs_ref, bfc_ref, out_ref):
    s = acc_ref[0] + acc_ref[1] + g2_ref[...]
    h = jnp.maximum(dinv_ref[:, :8] * s + b2_ref[...], 0.0)
    prod = h[None, :, :] * wfcs_ref[...]
    sums = jnp.sum(prod, axis=(1, 2))
    out_ref[...] = sums.reshape(1, 2) + bfc_ref[...]


def kernel(x, edge_index, w1, b1, w2, b2, wfc, bfc):
    ones_d = jnp.ones((CH, FD), jnp.float32)
    zeros8 = jnp.zeros((NP, FD), jnp.float32)
    zeros16 = jnp.zeros((NP, 16), jnp.float32)

    degp = _sc_degree(edge_index, ones_d, zeros8)

    g1, dinv16 = pl.pallas_call(
        _tc1_body,
        out_shape=(jax.ShapeDtypeStruct((NP, 16), jnp.float32),
                   jax.ShapeDtypeStruct((NP, 16), jnp.float32)),
    )(x, w1, degp)

    acc1 = _msgpass16(g1, edge_index, zeros16)

    g2 = pl.pallas_call(
        _tc2_body,
        out_shape=jax.ShapeDtypeStruct((NP, 8), jnp.float32),
    )(acc1, g1, dinv16, b1.reshape(1, 16), w2)

    acc2 = _msgpass8(g2, edge_index, zeros8)

    wfcs = jnp.pad(wfc.reshape(N, 8, 2).transpose(2, 0, 1),
                   ((0, 0), (0, NP - N), (0, 0)))
    out = pl.pallas_call(
        _tc3_body,
        out_shape=jax.ShapeDtypeStruct((1, 2), jnp.float32),
    )(acc2, g2, dinv16, b2.reshape(1, 8), wfcs, bfc.reshape(1, 2))
    return out


# speedup vs baseline: 106.8693x; 1.1130x over previous
import functools

import jax
import jax.numpy as jnp
from jax import lax
from jax.experimental import pallas as pl
from jax.experimental.pallas import tpu as pltpu
from jax.experimental.pallas import tpu_sc as plsc

N = 10000
NP = 10240
E = 640000
NC = 2
NS = 16
NW = NC * NS
E_PER_TILE = E // NW
CH = 4000
N_CHUNKS = E_PER_TILE // CH
N_PER_TILE = NP // NS
FD = 8


def _mesh():
    return plsc.VectorSubcoreMesh(core_axis_name="c", subcore_axis_name="s")


_SC_PARAMS = pltpu.CompilerParams(use_tc_tiling_on_sc=False)


@functools.partial(
    pl.kernel,
    out_type=jax.ShapeDtypeStruct((NC, NP, FD), jnp.float32),
    mesh=_mesh(),
    compiler_params=_SC_PARAMS,
    scratch_types=[
        pltpu.VMEM_SHARED((NP, FD), jnp.float32),
        pltpu.VMEM((CH,), jnp.int32),
        pltpu.VMEM((CH, FD), jnp.float32),
    ],
)
def _sc_degree(ei_hbm, ones_hbm, zeros_hbm, out_hbm, acc_sp, idx_v, ones_v):
    cid = lax.axis_index("c")
    sid = lax.axis_index("s")
    wid = cid * NS + sid
    rbase = sid * N_PER_TILE
    pltpu.sync_copy(zeros_hbm.at[pl.ds(rbase, N_PER_TILE)],
                    acc_sp.at[pl.ds(rbase, N_PER_TILE)])
    pltpu.sync_copy(ones_hbm, ones_v)
    plsc.subcore_barrier()

    def step(k, carry):
        base = wid * E_PER_TILE + k * CH
        pltpu.sync_copy(ei_hbm.at[1, pl.ds(base, CH)], idx_v)
        pltpu.sync_copy(ones_v, acc_sp.at[idx_v], add=True)
        return carry

    lax.fori_loop(0, N_CHUNKS, step, 0)
    plsc.subcore_barrier()
    pltpu.sync_copy(acc_sp.at[pl.ds(rbase, N_PER_TILE)],
                    out_hbm.at[cid, pl.ds(rbase, N_PER_TILE)])


def _make_msgpass(f):
    @functools.partial(
        pl.kernel,
        out_type=jax.ShapeDtypeStruct((NC, NP, f), jnp.float32),
        mesh=_mesh(),
        compiler_params=_SC_PARAMS,
        scratch_types=[
            pltpu.VMEM_SHARED((NP, f), jnp.float32),
            pltpu.VMEM_SHARED((NP, f), jnp.float32),
            pltpu.VMEM((2, CH), jnp.int32),
            pltpu.VMEM((CH, f), jnp.float32),
        ],
    )
    def _msgpass(g_hbm, ei_hbm, zeros_hbm, out_hbm,
                 g_sp, acc_sp, idx_v, msgs_v):
        cid = lax.axis_index("c")
        sid = lax.axis_index("s")
        wid = cid * NS + sid
        rbase = sid * N_PER_TILE
        pltpu.sync_copy(zeros_hbm.at[pl.ds(rbase, N_PER_TILE)],
                        acc_sp.at[pl.ds(rbase, N_PER_TILE)])
        pltpu.sync_copy(g_hbm.at[pl.ds(rbase, N_PER_TILE)],
                        g_sp.at[pl.ds(rbase, N_PER_TILE)])
        plsc.subcore_barrier()

        def step(k, carry):
            base = wid * E_PER_TILE + k * CH
            pltpu.sync_copy(ei_hbm.at[:, pl.ds(base, CH)], idx_v)
            pltpu.sync_copy(g_sp.at[idx_v.at[0]], msgs_v)
            pltpu.sync_copy(msgs_v, acc_sp.at[idx_v.at[1]], add=True)
            return carry

        lax.fori_loop(0, N_CHUNKS, step, 0)
        plsc.subcore_barrier()
        pltpu.sync_copy(acc_sp.at[pl.ds(rbase, N_PER_TILE)],
                        out_hbm.at[cid, pl.ds(rbase, N_PER_TILE)])

    return _msgpass


_msgpass16 = _make_msgpass(16)
_msgpass8 = _make_msgpass(8)


def _tc1_body(x_ref, w1_ref, degp_ref, g1_ref, dinv_ref):
    deg8 = degp_ref[0] + degp_ref[1] + 1.0
    dinv8 = lax.rsqrt(deg8)
    dinv16 = jnp.concatenate([dinv8, dinv8], axis=-1)
    h = jnp.dot(x_ref[...], w1_ref[...], preferred_element_type=jnp.float32)
    g1_ref[:N] = dinv16[:N] * h
    g1_ref[N:] = jnp.zeros((NP - N, 16), jnp.float32)
    dinv_ref[...] = dinv16


def _tc2_body(acc_ref, g1_ref, dinv_ref, b1_ref, w2_ref, g2_ref):
    s = acc_ref[0] + acc_ref[1] + g1_ref[...]
    h1 = jnp.maximum(dinv_ref[...] * s + b1_ref[...], 0.0)
    h2 = jnp.dot(h1, w2_ref[...], preferred_element_type=jnp.float32)
    g2_ref[...] = dinv_ref[:, :8] * h2


def _tc3_body(acc_ref, g2_ref, dinv_ref, b2_ref, wfcs_ref, bfc_ref, out_ref):
    s = acc_ref[0] + acc_ref[1] + g2_ref[...]
    h = jnp.maximum(dinv_ref[:, :8] * s + b2_ref[...], 0.0)
    prod = h[None, :, :] * wfcs_ref[...]
    sums = jnp.sum(prod, axis=(1, 2))
    out_ref[...] = sums.reshape(1, 2) + bfc_ref[...]


def kernel(x, edge_index, w1, b1, w2, b2, wfc, bfc):
    ones_d = jnp.ones((CH, FD), jnp.float32)
    zeros8 = jnp.zeros((NP, FD), jnp.float32)
    zeros16 = jnp.zeros((NP, 16), jnp.float32)

    degp = _sc_degree(edge_index, ones_d, zeros8)

    g1, dinv16 = pl.pallas_call(
        _tc1_body,
        out_shape=(jax.ShapeDtypeStruct((NP, 16), jnp.float32),
                   jax.ShapeDtypeStruct((NP, 16), jnp.float32)),
    )(x, w1, degp)

    acc1 = _msgpass16(g1, edge_index, zeros16)

    g2 = pl.pallas_call(
        _tc2_body,
        out_shape=jax.ShapeDtypeStruct((NP, 8), jnp.float32),
    )(acc1, g1, dinv16, b1.reshape(1, 16), w2)

    acc2 = _msgpass8(g2, edge_index, zeros8)

    wfcs = jnp.pad(wfc.reshape(N, 8, 2).transpose(2, 0, 1),
                   ((0, 0), (0, NP - N), (0, 0)))
    out = pl.pallas_call(
        _tc3_body,
        out_shape=jax.ShapeDtypeStruct((1, 2), jnp.float32),
    )(acc2, g2, dinv16, b2.reshape(1, 8), wfcs, bfc.reshape(1, 2))
    return out
